# baseline (speedup 1.0000x reference)
import jax
import jax.numpy as jnp
from jax.experimental import pallas as pl
from jax.experimental.pallas import tpu as pltpu

_D = 128
_K = 32
_NBLK = 4096
_BPB = 4
_CH = 2048


def _chunk(Xc, C, s, c2, ones_row):
    x2t = jax.lax.dot_general(ones_row, Xc * Xc, (((1,), (1,)), ((), ())),
                              preferred_element_type=jnp.float32)
    xct = jax.lax.dot_general(C, Xc, (((1,), (1,)), ((), ())),
                              preferred_element_type=jnp.float32)
    SL = s * (x2t - 2.0 * xct + c2)
    m = jnp.max(SL, axis=0, keepdims=True)
    e = jnp.exp(SL - m)
    A = e / jnp.sum(e, axis=0, keepdims=True)
    Ech = jax.lax.dot_general(A, Xc, (((1,), (0,)), ((), ())),
                              preferred_element_type=jnp.float32)
    asum = jnp.sum(A, axis=1, keepdims=True)
    return Ech, asum


def _enc_kernel(xt_ref, cw_ref, scale_ref, out_ref):
    C = cw_ref[...]
    s = scale_ref[...].reshape(_K, 1)
    c2 = jnp.sum(C * C, axis=1, keepdims=True)
    ones_row = jnp.ones((1, _D), dtype=jnp.float32)
    for i in range(_BPB):
        Ech = jnp.zeros((_K, _D), dtype=jnp.float32)
        asum = jnp.zeros((_K, 1), dtype=jnp.float32)
        for c in range(_NBLK // _CH):
            Xc = xt_ref[i, pl.ds(c * _CH, _CH), :]
            Ech_c, asum_c = _chunk(Xc, C, s, c2, ones_row)
            Ech = Ech + Ech_c
            asum = asum + asum_c
        out_ref[i] = Ech - asum * C


def kernel(x, codewords, scale):
    b, d, h, w = x.shape
    n_total = h * w
    xt = jnp.transpose(x, (0, 2, 3, 1)).reshape(b, n_total, d)
    s2 = scale.reshape(1, _K)
    out = pl.pallas_call(
        _enc_kernel,
        grid=(b // _BPB,),
        in_specs=[
            pl.BlockSpec((_BPB, _NBLK, _D), lambda bi: (bi, 0, 0)),
            pl.BlockSpec((_K, _D), lambda bi: (0, 0)),
            pl.BlockSpec((1, _K), lambda bi: (0, 0)),
        ],
        out_specs=pl.BlockSpec((_BPB, _K, _D), lambda bi: (bi, 0, 0)),
        out_shape=jax.ShapeDtypeStruct((b, _K, _D), jnp.float32),
        compiler_params=pltpu.CompilerParams(
            dimension_semantics=("arbitrary",),
        ),
    )(xt, codewords, s2)
    return out


# speedup vs baseline: 1.0640x; 1.0640x over previous
import jax
import jax.numpy as jnp
from jax.experimental import pallas as pl
from jax.experimental.pallas import tpu as pltpu

_D = 128
_K = 32
_NBLK = 4096
_BPB = 4


def _half(Xb, C, s, c2, ones_row):
    x2t = jax.lax.dot_general(ones_row, Xb * Xb, (((1,), (1,)), ((), ())),
                              preferred_element_type=jnp.float32)
    xct = jax.lax.dot_general(C, Xb, (((1,), (1,)), ((), ())),
                              preferred_element_type=jnp.float32)
    SL = s * (x2t - 2.0 * xct + c2)
    m = jnp.max(SL, axis=0, keepdims=True)
    e = jnp.exp(SL - m)
    A = e / jnp.sum(e, axis=0, keepdims=True)
    Ech = jax.lax.dot_general(A, Xb, (((1,), (0,)), ((), ())),
                              preferred_element_type=jnp.float32)
    asum = jnp.sum(A, axis=1, keepdims=True)
    return Ech, asum


def _enc_kernel(xt_ref, cw_ref, scale_ref, out_ref):
    C = cw_ref[...]
    s = scale_ref[...].reshape(_K, 1)
    c2 = jnp.sum(C * C, axis=1, keepdims=True)
    ones_row = jnp.ones((1, _D), dtype=jnp.float32)
    for i in range(_BPB):
        Ech, asum = _half(xt_ref[i], C, s, c2, ones_row)
        out_ref[i] = Ech - asum * C


def kernel(x, codewords, scale):
    b, d, h, w = x.shape
    n_total = h * w
    xt = jnp.transpose(x, (0, 2, 3, 1)).reshape(b, n_total, d)
    s2 = scale.reshape(1, _K)
    out = pl.pallas_call(
        _enc_kernel,
        grid=(b // _BPB,),
        in_specs=[
            pl.BlockSpec((_BPB, _NBLK, _D), lambda bi: (bi, 0, 0)),
            pl.BlockSpec((_K, _D), lambda bi: (0, 0)),
            pl.BlockSpec((1, _K), lambda bi: (0, 0)),
        ],
        out_specs=pl.BlockSpec((_BPB, _K, _D), lambda bi: (bi, 0, 0)),
        out_shape=jax.ShapeDtypeStruct((b, _K, _D), jnp.float32),
        compiler_params=pltpu.CompilerParams(
            dimension_semantics=("arbitrary",),
        ),
    )(xt, codewords, s2)
    return out
